# baseline (speedup 1.0000x reference)
import jax
import jax.numpy as jnp
from jax.experimental import pallas as pl
from jax.experimental.pallas import tpu as pltpu

_THRESHOLD = 0.5

_ROWS = 2 * 192 * 384
_W = 384
_BUF_ROWS = 8320

_CHUNKS = ([256, 512, 1024, 2048, 4096] + [8224] * 16
           + [4096, 2048, 1024, 512, 256])
assert sum(_CHUNKS) == _ROWS
_OFFS = [sum(_CHUNKS[:i]) for i in range(len(_CHUNKS))]
_NBUF = 4
_PD = 2


def _copy_ring(x_ref, o_ref, b0, b1, b2, b3, si0, si1, si2, si3,
               so0, so1, so2, so3):
    bufs = (b0, b1, b2, b3)
    in_sems = (si0, si1, si2, si3)
    out_sems = (so0, so1, so2, so3)
    n = len(_CHUNKS)

    def start_in(i):
        sz = _CHUNKS[i]
        return pltpu.async_copy(
            x_ref.at[pl.ds(_OFFS[i], sz)], bufs[i % _NBUF].at[pl.ds(0, sz)],
            in_sems[i % _NBUF])

    def start_out(i):
        sz = _CHUNKS[i]
        return pltpu.async_copy(
            bufs[i % _NBUF].at[pl.ds(0, sz)], o_ref.at[pl.ds(_OFFS[i], sz)],
            out_sems[i % _NBUF])

    in_copies = [None] * _NBUF
    out_copies = [None] * _NBUF
    for i in range(_PD):
        in_copies[i % _NBUF] = start_in(i)
    for i in range(n):
        b = i % _NBUF
        pf = i + _PD
        if pf < n:
            pb = pf % _NBUF
            if pf - _NBUF >= 0:
                out_copies[pb].wait()
            in_copies[pb] = start_in(pf)
        in_copies[b].wait()
        out_copies[b] = start_out(i)
    for c in out_copies:
        if c is not None:
            c.wait()


def kernel(x, heatmap0):
    del heatmap0
    b, c, h, w = x.shape
    x2 = x.reshape(_ROWS, _W)
    out = pl.pallas_call(
        _copy_ring,
        in_specs=[pl.BlockSpec(memory_space=pl.ANY)],
        out_specs=pl.BlockSpec(memory_space=pl.ANY),
        out_shape=jax.ShapeDtypeStruct((_ROWS, _W), x.dtype),
        scratch_shapes=(
            [pltpu.VMEM((_BUF_ROWS, _W), jnp.float32)] * _NBUF
            + [pltpu.SemaphoreType.DMA] * (2 * _NBUF)
        ),
    )(x2)
    return (out.reshape(b, c, h, w), jnp.float32(_THRESHOLD))


# speedup vs baseline: 1.0102x; 1.0102x over previous
import jax
import jax.numpy as jnp
from jax.experimental import pallas as pl
from jax.experimental.pallas import tpu as pltpu

_THRESHOLD = 0.5

_ROWS = 2 * 192 * 384
_W = 384
_BUF_ROWS = 8320

_CHUNKS = ([512, 1024, 2048, 4096] + [8256] * 16
           + [4096, 2048, 1024, 512])
assert sum(_CHUNKS) == _ROWS
_OFFS = [sum(_CHUNKS[:i]) for i in range(len(_CHUNKS))]
_NBUF = 4
_PD = 2


def _copy_ring(x_ref, o_ref, b0, b1, b2, b3, si0, si1, si2, si3,
               so0, so1, so2, so3):
    bufs = (b0, b1, b2, b3)
    in_sems = (si0, si1, si2, si3)
    out_sems = (so0, so1, so2, so3)
    n = len(_CHUNKS)

    def start_in(i):
        sz = _CHUNKS[i]
        return pltpu.async_copy(
            x_ref.at[pl.ds(_OFFS[i], sz)], bufs[i % _NBUF].at[pl.ds(0, sz)],
            in_sems[i % _NBUF])

    def start_out(i):
        sz = _CHUNKS[i]
        return pltpu.async_copy(
            bufs[i % _NBUF].at[pl.ds(0, sz)], o_ref.at[pl.ds(_OFFS[i], sz)],
            out_sems[i % _NBUF])

    in_copies = [None] * _NBUF
    out_copies = [None] * _NBUF
    for i in range(_PD):
        in_copies[i % _NBUF] = start_in(i)
    for i in range(n):
        b = i % _NBUF
        pf = i + _PD
        if pf < n:
            pb = pf % _NBUF
            if pf - _NBUF >= 0:
                out_copies[pb].wait()
            in_copies[pb] = start_in(pf)
        in_copies[b].wait()
        out_copies[b] = start_out(i)
    for c in out_copies:
        if c is not None:
            c.wait()


def kernel(x, heatmap0):
    del heatmap0
    b, c, h, w = x.shape
    x2 = x.reshape(_ROWS, _W)
    out = pl.pallas_call(
        _copy_ring,
        in_specs=[pl.BlockSpec(memory_space=pl.ANY)],
        out_specs=pl.BlockSpec(memory_space=pl.ANY),
        out_shape=jax.ShapeDtypeStruct((_ROWS, _W), x.dtype),
        scratch_shapes=(
            [pltpu.VMEM((_BUF_ROWS, _W), jnp.float32)] * _NBUF
            + [pltpu.SemaphoreType.DMA] * (2 * _NBUF)
        ),
    )(x2)
    return (out.reshape(b, c, h, w), jnp.float32(_THRESHOLD))


# speedup vs baseline: 1.0111x; 1.0010x over previous
import jax
import jax.numpy as jnp
from jax.experimental import pallas as pl
from jax.experimental.pallas import tpu as pltpu

_THRESHOLD = 0.5

_ROWS = 2 * 192 * 384
_W = 384
_BUF_ROWS = 8320

_CHUNKS = [1024, 2048, 4096] + [8320] * 16 + [4096, 2048, 1024]
assert sum(_CHUNKS) == _ROWS
_OFFS = [sum(_CHUNKS[:i]) for i in range(len(_CHUNKS))]
_NBUF = 4
_PD = 2


def _copy_ring(x_ref, o_ref, b0, b1, b2, b3, si0, si1, si2, si3,
               so0, so1, so2, so3):
    bufs = (b0, b1, b2, b3)
    in_sems = (si0, si1, si2, si3)
    out_sems = (so0, so1, so2, so3)
    n = len(_CHUNKS)

    def start_in(i):
        sz = _CHUNKS[i]
        return pltpu.async_copy(
            x_ref.at[pl.ds(_OFFS[i], sz)], bufs[i % _NBUF].at[pl.ds(0, sz)],
            in_sems[i % _NBUF])

    def start_out(i):
        sz = _CHUNKS[i]
        return pltpu.async_copy(
            bufs[i % _NBUF].at[pl.ds(0, sz)], o_ref.at[pl.ds(_OFFS[i], sz)],
            out_sems[i % _NBUF])

    in_copies = [None] * _NBUF
    out_copies = [None] * _NBUF
    for i in range(_PD):
        in_copies[i % _NBUF] = start_in(i)
    for i in range(n):
        b = i % _NBUF
        pf = i + _PD
        if pf < n:
            pb = pf % _NBUF
            if pf - _NBUF >= 0:
                out_copies[pb].wait()
            in_copies[pb] = start_in(pf)
        in_copies[b].wait()
        out_copies[b] = start_out(i)
    for c in out_copies:
        if c is not None:
            c.wait()


def kernel(x, heatmap0):
    del heatmap0
    b, c, h, w = x.shape
    x2 = x.reshape(_ROWS, _W)
    out = pl.pallas_call(
        _copy_ring,
        in_specs=[pl.BlockSpec(memory_space=pl.ANY)],
        out_specs=pl.BlockSpec(memory_space=pl.ANY),
        out_shape=jax.ShapeDtypeStruct((_ROWS, _W), x.dtype),
        scratch_shapes=(
            [pltpu.VMEM((_BUF_ROWS, _W), jnp.float32)] * _NBUF
            + [pltpu.SemaphoreType.DMA] * (2 * _NBUF)
        ),
    )(x2)
    return (out.reshape(b, c, h, w), jnp.float32(_THRESHOLD))
